# Initial kernel scaffold; baseline (speedup 1.0000x reference)
import jax
import jax.numpy as jnp
from jax.experimental import pallas as pl


def kernel(h, graph_ids, W1a, b1a, W1b, b1b, W2a, b2a, W2b, b2b):
    raise NotImplementedError("write your pallas kernel here")


# speedup vs baseline: 5.6199x; 5.6199x over previous
import functools

import jax
import jax.numpy as jnp
from jax import lax
from jax.experimental import pallas as pl
from jax.experimental.pallas import tpu as pltpu

N = 100000
D = 128
G = 1024
B = 2000
W = 64
NBLK = N // B


def _fused_kernel(first_ref, nwin_ref, h_ref, ids_ref,
                  w1a_ref, b1a_ref, w1b_ref, b1b_ref,
                  w2a_ref, b2a_ref, w2b_ref, b2b_ref,
                  out_ref, acc_ref):
    i = pl.program_id(0)

    @pl.when(i == 0)
    def _init():
        acc_ref[...] = jnp.zeros_like(acc_ref)

    x = jnp.dot(h_ref[...], w1a_ref[...], preferred_element_type=jnp.float32)
    x = jnp.maximum(x + b1a_ref[...], 0.0)
    act = jnp.dot(x, w1b_ref[...], preferred_element_type=jnp.float32)
    act = jnp.maximum(act + b1b_ref[...], 0.0)

    ids = ids_ref[0, 0, :]
    first = first_ref[i]
    nwin = nwin_ref[i]

    col_iota = lax.broadcasted_iota(jnp.int32, (B, W), 1)

    def body(k, carry):
        base = first + k * W
        rel = ids - base
        oh = (rel[:, None] == col_iota).astype(jnp.float32)
        partial = lax.dot_general(
            oh, act, (((0,), (0,)), ((), ())),
            preferred_element_type=jnp.float32)
        acc_ref[pl.ds(base, W), :] += partial
        return carry

    lax.fori_loop(0, nwin, body, 0)

    @pl.when(i == NBLK - 1)
    def _finish():
        pooled = acc_ref[0:G, :]
        y = jnp.dot(pooled, w2a_ref[...], preferred_element_type=jnp.float32)
        y = jnp.maximum(y + b2a_ref[...], 0.0)
        z = jnp.dot(y, w2b_ref[...], preferred_element_type=jnp.float32)
        out_ref[...] = jnp.maximum(z + b2b_ref[...], 0.0)


@jax.jit
def kernel(h, graph_ids, W1a, b1a, W1b, b1b, W2a, b2a, W2b, b2b):
    ids32 = graph_ids.astype(jnp.int32)
    ids3 = ids32.reshape(NBLK, 1, B)
    firsts = ids32[::B]
    lasts = ids32[B - 1::B]
    nwin = (lasts - firsts) // W + 1

    full = lambda shape: pl.BlockSpec(shape, lambda i, *_: (0,) * len(shape))
    row = lambda: pl.BlockSpec((1, D), lambda i, *_: (0, 0))

    grid_spec = pltpu.PrefetchScalarGridSpec(
        num_scalar_prefetch=2,
        grid=(NBLK,),
        in_specs=[
            pl.BlockSpec((B, D), lambda i, *_: (i, 0)),
            pl.BlockSpec((1, 1, B), lambda i, *_: (i, 0, 0)),
            full((D, D)), row(), full((D, D)), row(),
            full((D, D)), row(), full((D, D)), row(),
        ],
        out_specs=pl.BlockSpec((G, D), lambda i, *_: (0, 0)),
        scratch_shapes=[pltpu.VMEM((G + W, D), jnp.float32)],
    )

    return pl.pallas_call(
        _fused_kernel,
        grid_spec=grid_spec,
        out_shape=jax.ShapeDtypeStruct((G, D), jnp.float32),
    )(firsts, nwin, h, ids3,
      W1a, b1a.reshape(1, D), W1b, b1b.reshape(1, D),
      W2a, b2a.reshape(1, D), W2b, b2b.reshape(1, D))
